# baseline (speedup 1.0000x reference)
import jax
import jax.numpy as jnp
from jax.experimental import pallas as pl
from jax.experimental.pallas import tpu as pltpu

_N = 128
_H = 1024
_HEADS = 8
_SUB = _H // _HEADS
_BJ = 8
_NJ = _N // _BJ
_R = _N * _BJ


def _body(x_ref, y_ref, Wu_ref, bu_ref, Wp_ref, bp_ref, wa1bd_ref, wa3bd_ref,
          b8_ref, Wm_ref, bm_ref, Wg_ref, bg_ref, gamma_ref, beta_ref,
          out_ref, w_ref,
          u_s, a_s, E_s, p_s):
    k = pl.program_id(0)

    @pl.when(k == 0)
    def _init():
        u = jnp.dot(x_ref[...], Wu_ref[...], preferred_element_type=jnp.float32)
        u = jnp.maximum(u + bu_ref[...], 0.0)
        u_s[...] = u
        a_s[...] = jnp.dot(u, wa1bd_ref[...], preferred_element_type=jnp.float32)

    par = jax.lax.rem(k, 2)

    yb = y_ref[...].reshape(_R, _H)
    p2 = jnp.dot(yb, Wp_ref[...], preferred_element_type=jnp.float32)
    p2 = jnp.maximum(p2 + bp_ref[...], 0.0)
    p_s[pl.ds(par * _R, _R), :] = p2

    u_full = u_s[...]
    a_full = a_s[...]

    def consume(blk, pprev):
        c8 = jnp.dot(pprev, wa3bd_ref[...], preferred_element_type=jnp.float32)
        l3 = c8.reshape(_N, _BJ, _HEADS) + a_full[:, None, :]
        m = jnp.max(l3, axis=0)
        e3 = jnp.exp(l3 - m[None, :, :])
        s = jnp.sum(e3, axis=0)
        w3 = e3 * (1.0 / s)[None, :, :]
        w_ref[:, pl.ds(blk * _BJ, _BJ), :] = w3
        w_exp = jnp.dot(w3.reshape(_R, _HEADS), b8_ref[...],
                        preferred_element_type=jnp.float32).reshape(_N, _BJ, _H)
        p3 = pprev.reshape(_N, _BJ, _H)
        contrib = jnp.sum(w_exp * p3 * u_full[:, None, :], axis=0)
        E_s[pl.ds(blk * _BJ, _BJ), :] = contrib

    @pl.when(k > 0)
    def _consume_prev():
        prev_par = jax.lax.rem(k + 1, 2)
        consume(k - 1, p_s[pl.ds(prev_par * _R, _R), :])

    @pl.when(k == _NJ - 1)
    def _consume_last():
        consume(k, p_s[pl.ds(par * _R, _R), :])

        cols = []
        for h in range(_HEADS):
            sl = slice(h * _SUB, (h + 1) * _SUB)
            agg_h = jnp.dot(E_s[:, sl], Wm_ref[h], preferred_element_type=jnp.float32)
            cols.append(jnp.maximum(agg_h + bm_ref[h:h + 1, :], 0.0))
        ra = jnp.concatenate(cols, axis=1)
        fin = jnp.dot(ra, Wg_ref[...], preferred_element_type=jnp.float32)
        z = x_ref[...] + fin + bg_ref[...]
        mu = jnp.mean(z, axis=1, keepdims=True)
        d = z - mu
        var = jnp.mean(d * d, axis=1, keepdims=True)
        out_ref[...] = d * jax.lax.rsqrt(var + 1e-5) * gamma_ref[...] + beta_ref[...]


def kernel(x, y, Wu, bu, Wp, bp, Wa, ba, Wm, bm, Wg, bg, gamma, beta):
    del ba
    wa1 = Wa[:, :_SUB, 0]
    wa3 = Wa[:, 2 * _SUB:, 0]
    hid = jnp.arange(_H, dtype=jnp.int32) // _SUB
    onehot = (hid[:, None] == jnp.arange(_HEADS, dtype=jnp.int32)[None, :])
    wa1bd = jnp.where(onehot, wa1.reshape(_H)[:, None], 0.0).astype(jnp.float32)
    wa3bd = jnp.where(onehot, wa3.reshape(_H)[:, None], 0.0).astype(jnp.float32)
    b8 = onehot.astype(jnp.float32).T
    row = lambda v: v.reshape(1, _H)

    out, w = pl.pallas_call(
        _body,
        grid=(_NJ,),
        in_specs=[
            pl.BlockSpec((_N, _H), lambda k: (0, 0)),
            pl.BlockSpec((_N, _BJ, _H), lambda k: (0, k, 0)),
            pl.BlockSpec((_H, _H), lambda k: (0, 0)),
            pl.BlockSpec((1, _H), lambda k: (0, 0)),
            pl.BlockSpec((_H, _H), lambda k: (0, 0)),
            pl.BlockSpec((1, _H), lambda k: (0, 0)),
            pl.BlockSpec((_H, _HEADS), lambda k: (0, 0)),
            pl.BlockSpec((_H, _HEADS), lambda k: (0, 0)),
            pl.BlockSpec((_HEADS, _H), lambda k: (0, 0)),
            pl.BlockSpec((_HEADS, _SUB, _SUB), lambda k: (0, 0, 0)),
            pl.BlockSpec((_HEADS, _SUB), lambda k: (0, 0)),
            pl.BlockSpec((_H, _H), lambda k: (0, 0)),
            pl.BlockSpec((1, _H), lambda k: (0, 0)),
            pl.BlockSpec((1, _H), lambda k: (0, 0)),
            pl.BlockSpec((1, _H), lambda k: (0, 0)),
        ],
        out_specs=[
            pl.BlockSpec((_N, _H), lambda k: (0, 0)),
            pl.BlockSpec((_N, _N, _HEADS), lambda k: (0, 0, 0)),
        ],
        out_shape=[
            jax.ShapeDtypeStruct((_N, _H), jnp.float32),
            jax.ShapeDtypeStruct((_N, _N, _HEADS), jnp.float32),
        ],
        scratch_shapes=[
            pltpu.VMEM((_N, _H), jnp.float32),
            pltpu.VMEM((_N, _HEADS), jnp.float32),
            pltpu.VMEM((_N, _H), jnp.float32),
            pltpu.VMEM((2 * _R, _H), jnp.float32),
        ],
        compiler_params=pltpu.CompilerParams(
            dimension_semantics=("arbitrary",),
        ),
    )(x, y, Wu, row(bu), Wp, row(bp), wa1bd, wa3bd, b8, Wm, bm, Wg, row(bg),
      row(gamma), row(beta))
    return (out, jnp.transpose(w, (2, 0, 1))[..., None])


# speedup vs baseline: 1.0987x; 1.0987x over previous
import jax
import jax.numpy as jnp
from jax.experimental import pallas as pl
from jax.experimental.pallas import tpu as pltpu

_N = 128
_H = 1024
_HEADS = 8
_SUB = _H // _HEADS
_BJ = 16
_BJS = 16
_NBLK = _N // _BJ


def _body(x_ref, y_ref, Wu_ref, bu_ref, Wp_ref, bp_ref, wa1bd_ref, wa3bd_ref,
          b8_ref, Wm_ref, bm_ref, Wg_ref, bg_ref, gamma_ref, beta_ref,
          out_ref, w_ref,
          u_s, a_s, E_s):
    k = pl.program_id(0)

    @pl.when(k == 0)
    def _init():
        u = jnp.dot(x_ref[...], Wu_ref[...], preferred_element_type=jnp.float32)
        u = jnp.maximum(u + bu_ref[...], 0.0)
        u_s[...] = u
        a_s[...] = jnp.dot(u, wa1bd_ref[...], preferred_element_type=jnp.float32)

    u_full = u_s[...]
    a_full = a_s[...]
    for sub in range(_BJ // _BJS):
        j0 = sub * _BJS
        yb = y_ref[:, j0:j0 + _BJS, :].reshape(_N * _BJS, _H)
        p2 = jnp.dot(yb, Wp_ref[...], preferred_element_type=jnp.float32)
        p2 = jnp.maximum(p2 + bp_ref[...], 0.0)

        c8 = jnp.dot(p2, wa3bd_ref[...], preferred_element_type=jnp.float32)
        l3 = c8.reshape(_N, _BJS, _HEADS) + a_full[:, None, :]

        m = jnp.max(l3, axis=0)
        e3 = jnp.exp(l3 - m[None, :, :])
        s = jnp.sum(e3, axis=0)
        w3 = e3 * (1.0 / s)[None, :, :]
        w_ref[:, j0:j0 + _BJS, :] = w3

        w_exp = jnp.dot(w3.reshape(_N * _BJS, _HEADS), b8_ref[...],
                        preferred_element_type=jnp.float32).reshape(_N, _BJS, _H)
        p3 = p2.reshape(_N, _BJS, _H)
        contrib = jnp.sum(w_exp * p3 * u_full[:, None, :], axis=0)
        E_s[pl.ds(k * _BJ + j0, _BJS), :] = contrib

    @pl.when(k == _NBLK - 1)
    def _finalize():
        cols = []
        for h in range(_HEADS):
            sl = slice(h * _SUB, (h + 1) * _SUB)
            agg_h = jnp.dot(E_s[:, sl], Wm_ref[h], preferred_element_type=jnp.float32)
            cols.append(jnp.maximum(agg_h + bm_ref[h:h + 1, :], 0.0))
        ra = jnp.concatenate(cols, axis=1)
        fin = jnp.dot(ra, Wg_ref[...], preferred_element_type=jnp.float32)
        z = x_ref[...] + fin + bg_ref[...]
        mu = jnp.mean(z, axis=1, keepdims=True)
        d = z - mu
        var = jnp.mean(d * d, axis=1, keepdims=True)
        out_ref[...] = d * jax.lax.rsqrt(var + 1e-5) * gamma_ref[...] + beta_ref[...]


def kernel(x, y, Wu, bu, Wp, bp, Wa, ba, Wm, bm, Wg, bg, gamma, beta):
    del ba
    wa1 = Wa[:, :_SUB, 0]
    wa3 = Wa[:, 2 * _SUB:, 0]
    hid = jnp.arange(_H, dtype=jnp.int32) // _SUB
    onehot = (hid[:, None] == jnp.arange(_HEADS, dtype=jnp.int32)[None, :])
    wa1bd = jnp.where(onehot, wa1.reshape(_H)[:, None], 0.0).astype(jnp.float32)
    wa3bd = jnp.where(onehot, wa3.reshape(_H)[:, None], 0.0).astype(jnp.float32)
    b8 = onehot.astype(jnp.float32).T
    row = lambda v: v.reshape(1, _H)

    out, w = pl.pallas_call(
        _body,
        grid=(_NBLK,),
        in_specs=[
            pl.BlockSpec((_N, _H), lambda k: (0, 0)),
            pl.BlockSpec((_N, _BJ, _H), lambda k: (0, k, 0)),
            pl.BlockSpec((_H, _H), lambda k: (0, 0)),
            pl.BlockSpec((1, _H), lambda k: (0, 0)),
            pl.BlockSpec((_H, _H), lambda k: (0, 0)),
            pl.BlockSpec((1, _H), lambda k: (0, 0)),
            pl.BlockSpec((_H, _HEADS), lambda k: (0, 0)),
            pl.BlockSpec((_H, _HEADS), lambda k: (0, 0)),
            pl.BlockSpec((_HEADS, _H), lambda k: (0, 0)),
            pl.BlockSpec((_HEADS, _SUB, _SUB), lambda k: (0, 0, 0)),
            pl.BlockSpec((_HEADS, _SUB), lambda k: (0, 0)),
            pl.BlockSpec((_H, _H), lambda k: (0, 0)),
            pl.BlockSpec((1, _H), lambda k: (0, 0)),
            pl.BlockSpec((1, _H), lambda k: (0, 0)),
            pl.BlockSpec((1, _H), lambda k: (0, 0)),
        ],
        out_specs=[
            pl.BlockSpec((_N, _H), lambda k: (0, 0)),
            pl.BlockSpec((_N, _BJ, _HEADS), lambda k: (0, k, 0)),
        ],
        out_shape=[
            jax.ShapeDtypeStruct((_N, _H), jnp.float32),
            jax.ShapeDtypeStruct((_N, _N, _HEADS), jnp.float32),
        ],
        scratch_shapes=[
            pltpu.VMEM((_N, _H), jnp.float32),
            pltpu.VMEM((_N, _HEADS), jnp.float32),
            pltpu.VMEM((_N, _H), jnp.float32),
        ],
        compiler_params=pltpu.CompilerParams(
            dimension_semantics=("arbitrary",),
        ),
    )(x, y, Wu, row(bu), Wp, row(bp), wa1bd, wa3bd, b8, Wm, bm, Wg, row(bg),
      row(gamma), row(beta))
    return (out, jnp.transpose(w, (2, 0, 1))[..., None])
